# baseline (speedup 1.0000x reference)
import functools

import jax
import jax.numpy as jnp
from jax import lax
from jax.experimental import pallas as pl
from jax.experimental.pallas import tpu as pltpu
from jax.experimental.pallas import tpu_sc as plsc

_NW = 32
_CHUNK = 20480


def _sc_histogram(idx_flat, L, B, V_pad):
    mesh = plsc.VectorSubcoreMesh(core_axis_name="c", subcore_axis_name="s")
    cols_per_tile = (L + _NW - 1) // _NW
    inv_b = 1.0 / float(B)
    n_chunks = V_pad // _CHUNK

    @functools.partial(
        pl.kernel,
        out_type=jax.ShapeDtypeStruct((L * V_pad // 2,), jnp.float32),
        mesh=mesh,
        scratch_types=[
            pltpu.VMEM((V_pad,), jnp.float32),
            pltpu.VMEM((B,), jnp.int32),
            pltpu.VMEM((_CHUNK // 2,), jnp.float32),
            pltpu.SemaphoreType.DMA,
        ],
        compiler_params=pltpu.CompilerParams(needs_layout_passes=False),
    )
    def hist_kernel(idx_hbm, out_hbm, hist_v, idx_v, pack_v, sem0):
        cid = lax.axis_index("c")
        sid = lax.axis_index("s")
        wid = sid * 2 + cid
        ones = jnp.full((16,), inv_b, dtype=jnp.float32)
        zeros16 = jnp.zeros((16,), dtype=jnp.float32)

        def permute(v):
            g = v & 31
            return (v - g) + (g >> 1) + ((v & 1) << 4)

        for t in range(cols_per_tile):
            col = t * _NW + wid

            @pl.when(col < L)
            def _():
                icp = pltpu.async_copy(idx_hbm.at[pl.ds(col * B, B)], idx_v, sem0)

                if t == 0:
                    def zbody(i, carry):
                        hist_v[pl.ds(i * 16, 16)] = zeros16
                        return carry

                    lax.fori_loop(0, V_pad // 16, zbody, 0, unroll=8)

                icp.wait()

                def body(i, carry):
                    iv = permute(idx_v[pl.ds(i * 16, 16)])
                    plsc.addupdate_scatter(hist_v, [iv], ones)
                    return carry

                lax.fori_loop(0, B // 16, body, 0, unroll=8)

                for c in range(n_chunks):
                    def pbody(i, carry):
                        base = c * _CHUNK + i * 32
                        a = hist_v[pl.ds(base, 16)]
                        b = hist_v[pl.ds(base + 16, 16)]
                        pack_v[pl.ds(i * 16, 16)] = plsc.bitcast(
                            plsc.pack(a, b, format=plsc.PackFormat.INTERLEAVED),
                            jnp.float32)
                        return carry

                    lax.fori_loop(0, _CHUNK // 32, pbody, 0, unroll=8)
                    pltpu.sync_copy(
                        pack_v,
                        out_hbm.at[pl.ds(col * (V_pad // 2) + c * (_CHUNK // 2),
                                         _CHUNK // 2)])

                if t + 1 < cols_per_tile:
                    def zsbody(i, carry):
                        iv = permute(idx_v[pl.ds(i * 16, 16)])
                        plsc.store_scatter(hist_v, [iv], zeros16)
                        return carry

                    lax.fori_loop(0, B // 16, zsbody, 0, unroll=8)

    return hist_kernel(idx_flat)


def _tc_mean(weights, emb, L, V, E, VB):
    V_pad = weights.shape[1]
    K = V_pad // VB

    def body(w_ref, e_ref, o_ref):
        k = pl.program_id(0)
        w = w_ref[...]
        e = e_ref[...]
        row = k * VB + lax.broadcasted_iota(jnp.int32, (VB, E), 0)
        e = jnp.where(row < V, e, 0.0).astype(jnp.bfloat16)
        acc = lax.dot_general(w, e, (((1,), (0,)), ((), ())),
                              preferred_element_type=jnp.float32)

        @pl.when(k == 0)
        def _():
            o_ref[...] = jnp.zeros_like(o_ref)

        o_ref[...] += acc

    return pl.pallas_call(
        body,
        grid=(K,),
        in_specs=[
            pl.BlockSpec((L, VB), lambda k: (0, k)),
            pl.BlockSpec((VB, E), lambda k: (k, 0)),
        ],
        out_specs=pl.BlockSpec((L, E), lambda k: (0, 0)),
        out_shape=jax.ShapeDtypeStruct((L, E), jnp.float32),
        compiler_params=pltpu.CompilerParams(
            dimension_semantics=("arbitrary",)),
    )(weights, emb)


def _tc_linear(mean, fc_w, fc_b2d, L, V, E, VB):
    K = pl.cdiv(V, VB)

    def body(m_ref, w_ref, b_ref, o_ref):
        m = m_ref[...]
        w = w_ref[...]
        b = b_ref[...]
        o_ref[...] = lax.dot_general(m, w, (((1,), (1,)), ((), ())),
                                     preferred_element_type=jnp.float32) + b

    return pl.pallas_call(
        body,
        grid=(K,),
        in_specs=[
            pl.BlockSpec((L, E), lambda k: (0, 0)),
            pl.BlockSpec((VB, E), lambda k: (k, 0)),
            pl.BlockSpec((1, VB), lambda k: (0, k)),
        ],
        out_specs=pl.BlockSpec((L, VB), lambda k: (0, k)),
        out_shape=jax.ShapeDtypeStruct((L, V), jnp.float32),
        compiler_params=pltpu.CompilerParams(
            dimension_semantics=("parallel",)),
    )(mean, fc_w, fc_b2d)


def kernel(context_word_idx, embedding, fc_w, fc_b):
    B, L = context_word_idx.shape
    V, E = embedding.shape
    VB2 = 4096
    VB3 = 16384
    V_pad = ((V + VB2 - 1) // VB2) * VB2
    idx = context_word_idx.astype(jnp.int32)
    idx_flat = idx.T.reshape(-1)
    packed = _sc_histogram(idx_flat, L, B, V_pad).reshape(L, V_pad // 2)
    weights = lax.bitcast_convert_type(packed, jnp.bfloat16).reshape(L, V_pad)
    mean = _tc_mean(weights, embedding, L, V, E, VB2)
    out = _tc_linear(mean, fc_w, fc_b.reshape(1, V), L, V, E, VB3)
    return out


# speedup vs baseline: 2.0335x; 2.0335x over previous
import functools

import jax
import jax.numpy as jnp
from jax import lax
from jax.experimental import pallas as pl
from jax.experimental.pallas import tpu as pltpu
from jax.experimental.pallas import tpu_sc as plsc

_NW = 32
_CHUNK = 20480


def _sc_histogram(idx_flat, L, B, V_pad):
    mesh = plsc.VectorSubcoreMesh(core_axis_name="c", subcore_axis_name="s")
    cols_per_tile = (L + _NW - 1) // _NW
    inv_b = 1.0 / float(B)
    n_chunks = V_pad // _CHUNK

    @functools.partial(
        pl.kernel,
        out_type=jax.ShapeDtypeStruct((L * V_pad // 2,), jnp.float32),
        mesh=mesh,
        scratch_types=[
            pltpu.VMEM((V_pad,), jnp.float32),
            pltpu.VMEM((B,), jnp.int32),
            pltpu.VMEM((_CHUNK // 2,), jnp.float32),
            pltpu.SemaphoreType.DMA,
        ],
        compiler_params=pltpu.CompilerParams(needs_layout_passes=False),
    )
    def hist_kernel(idx_hbm, out_hbm, hist_v, idx_v, pack_v, sem0):
        cid = lax.axis_index("c")
        sid = lax.axis_index("s")
        wid = sid * 2 + cid
        ones = jnp.full((16,), inv_b, dtype=jnp.float32)
        zeros16 = jnp.zeros((16,), dtype=jnp.float32)

        Vh = V_pad // 2

        def permute(v):
            ishi = v >= Vh
            u = jnp.where(ishi, v - Vh, v)
            g = u & 15
            return (u - g) * 2 + g + jnp.where(ishi, 16, 0)

        for t in range(cols_per_tile):
            col = t * _NW + wid

            @pl.when(col < L)
            def _():
                icp = pltpu.async_copy(idx_hbm.at[pl.ds(col * B, B)], idx_v, sem0)

                if t == 0:
                    def zbody(i, carry):
                        hist_v[pl.ds(i * 16, 16)] = zeros16
                        return carry

                    lax.fori_loop(0, V_pad // 16, zbody, 0, unroll=8)

                icp.wait()

                def body(i, carry):
                    iv = permute(idx_v[pl.ds(i * 16, 16)])
                    plsc.addupdate_scatter(hist_v, [iv], ones)
                    return carry

                lax.fori_loop(0, B // 16, body, 0, unroll=8)

                for c in range(n_chunks):
                    def pbody(i, carry):
                        base = c * _CHUNK + i * 32
                        a = hist_v[pl.ds(base, 16)]
                        b = hist_v[pl.ds(base + 16, 16)]
                        pack_v[pl.ds(i * 16, 16)] = plsc.bitcast(
                            plsc.pack(a, b, format=plsc.PackFormat.INTERLEAVED),
                            jnp.float32)
                        return carry

                    lax.fori_loop(0, _CHUNK // 32, pbody, 0, unroll=8)
                    pltpu.sync_copy(
                        pack_v,
                        out_hbm.at[pl.ds(col * (V_pad // 2) + c * (_CHUNK // 2),
                                         _CHUNK // 2)])

                if t + 1 < cols_per_tile:
                    def zsbody(i, carry):
                        iv = permute(idx_v[pl.ds(i * 16, 16)])
                        plsc.store_scatter(hist_v, [iv], zeros16)
                        return carry

                    lax.fori_loop(0, B // 16, zsbody, 0, unroll=8)

    return hist_kernel(idx_flat)


def _tc_mean(packed, emb, L, V, E, VBw):
    Wn = packed.shape[1]
    K = Wn // VBw
    Vh = Wn
    KHI_MAX = (V - 1) // VBw

    def body(p_ref, elo_ref, ehi_ref, o_ref):
        k = pl.program_id(0)
        wi = lax.bitcast_convert_type(p_ref[...], jnp.uint32)
        lo = lax.bitcast_convert_type(wi << 16, jnp.float32)
        hi = lax.bitcast_convert_type(wi & jnp.uint32(0xFFFF0000), jnp.float32)
        elo = elo_ref[...]
        ehi = ehi_ref[...]
        row = Vh + k * VBw + lax.broadcasted_iota(jnp.int32, (VBw, E), 0)
        ehi = jnp.where(row < V, ehi, 0.0)
        acc = (lax.dot_general(lo, elo, (((1,), (0,)), ((), ())),
                               preferred_element_type=jnp.float32) +
               lax.dot_general(hi, ehi, (((1,), (0,)), ((), ())),
                               preferred_element_type=jnp.float32))

        @pl.when(k == 0)
        def _():
            o_ref[...] = jnp.zeros_like(o_ref)

        o_ref[...] += acc

    return pl.pallas_call(
        body,
        grid=(K,),
        in_specs=[
            pl.BlockSpec((L, VBw), lambda k: (0, k)),
            pl.BlockSpec((VBw, E), lambda k: (k, 0)),
            pl.BlockSpec((VBw, E), lambda k: (jnp.minimum(k + K, KHI_MAX), 0)),
        ],
        out_specs=pl.BlockSpec((L, E), lambda k: (0, 0)),
        out_shape=jax.ShapeDtypeStruct((L, E), jnp.float32),
        compiler_params=pltpu.CompilerParams(
            dimension_semantics=("arbitrary",)),
    )(packed, emb, emb)


def _tc_linear(mean, fc_w, fc_b2d, L, V, E, VB):
    K = pl.cdiv(V, VB)

    def body(m_ref, w_ref, b_ref, o_ref):
        m = m_ref[...]
        w = w_ref[...]
        b = b_ref[...]
        o_ref[...] = lax.dot_general(m, w, (((1,), (1,)), ((), ())),
                                     preferred_element_type=jnp.float32) + b

    return pl.pallas_call(
        body,
        grid=(K,),
        in_specs=[
            pl.BlockSpec((L, E), lambda k: (0, 0)),
            pl.BlockSpec((VB, E), lambda k: (k, 0)),
            pl.BlockSpec((1, VB), lambda k: (0, k)),
        ],
        out_specs=pl.BlockSpec((L, VB), lambda k: (0, k)),
        out_shape=jax.ShapeDtypeStruct((L, V), jnp.float32),
        compiler_params=pltpu.CompilerParams(
            dimension_semantics=("parallel",)),
    )(mean, fc_w, fc_b2d)


def kernel(context_word_idx, embedding, fc_w, fc_b):
    B, L = context_word_idx.shape
    V, E = embedding.shape
    VB2 = 4096
    VB3 = 16384
    V_pad = ((V + VB2 - 1) // VB2) * VB2
    idx = context_word_idx.astype(jnp.int32)
    idx_flat = idx.T.reshape(-1)
    packed = _sc_histogram(idx_flat, L, B, V_pad).reshape(L, V_pad // 2)
    mean = _tc_mean(packed, embedding, L, V, E, VB2 // 2)
    out = _tc_linear(mean, fc_w, fc_b.reshape(1, V), L, V, E, VB3)
    return out
